# Initial kernel scaffold; baseline (speedup 1.0000x reference)
import jax
import jax.numpy as jnp
from jax.experimental import pallas as pl


def kernel(inp, table):
    raise NotImplementedError("write your pallas kernel here")


# speedup vs baseline: 1.5734x; 1.5734x over previous
import functools

import jax
import jax.numpy as jnp
from jax import lax
from jax.experimental import pallas as pl
from jax.experimental.pallas import tpu as pltpu
from jax.experimental.pallas import tpu_sc as plsc

NC = 2
NS = 16
NW = NC * NS
CHUNK = 128


@functools.lru_cache(maxsize=None)
def _make_gather(n, width):
    assert n % (CHUNK * NW) == 0
    b_per_w = n // NW
    n_steps = b_per_w // CHUNK
    mesh = plsc.VectorSubcoreMesh(core_axis_name="c", subcore_axis_name="s")

    @functools.partial(
        pl.kernel,
        mesh=mesh,
        out_type=jax.ShapeDtypeStruct((n, width), jnp.float32),
        scratch_types=[
            pltpu.VMEM((CHUNK,), jnp.int32),
            pltpu.VMEM((CHUNK, width), jnp.float32),
            pltpu.SemaphoreType.DMA,
        ],
        compiler_params=pltpu.CompilerParams(use_tc_tiling_on_sc=False),
    )
    def gather_kernel(idx_hbm, table_hbm, out_hbm, idx_v, rows_v, sem):
        wid = lax.axis_index("s") * NC + lax.axis_index("c")
        base = wid * b_per_w

        def step(j, carry):
            off = base + j * CHUNK
            pltpu.sync_copy(idx_hbm.at[pl.ds(off, CHUNK)], idx_v)
            pltpu.async_copy(table_hbm.at[idx_v], rows_v, sem).wait()
            pltpu.sync_copy(rows_v, out_hbm.at[pl.ds(off, CHUNK)])
            return carry

        lax.fori_loop(0, n_steps, step, 0)

    return gather_kernel


def kernel(inp, table):
    b, h = inp.shape
    _, width = table.shape
    idx = inp.reshape(b * h).astype(jnp.int32)
    out = _make_gather(b * h, width)(idx, table)
    return out.reshape(b, h, width)


# speedup vs baseline: 1.8797x; 1.1947x over previous
import functools

import jax
import jax.numpy as jnp
from jax import lax
from jax.experimental import pallas as pl
from jax.experimental.pallas import tpu as pltpu
from jax.experimental.pallas import tpu_sc as plsc

NC = 2
NS = 16
NW = NC * NS
CHUNK = 128
NBUF = 4


@functools.lru_cache(maxsize=None)
def _make_gather(n, width):
    assert n % (CHUNK * NW) == 0
    b_per_w = n // NW
    n_steps = b_per_w // CHUNK
    n_outer = (n_steps + NBUF - 1) // NBUF
    mesh = plsc.VectorSubcoreMesh(core_axis_name="c", subcore_axis_name="s")

    @functools.partial(
        pl.kernel,
        mesh=mesh,
        out_type=jax.ShapeDtypeStruct((n, width), jnp.float32),
        scratch_types=[
            pltpu.VMEM((n_steps, CHUNK), jnp.int32),
            *([pltpu.VMEM((CHUNK, width), jnp.float32)] * NBUF),
            *([pltpu.SemaphoreType.DMA] * NBUF),
        ],
        compiler_params=pltpu.CompilerParams(use_tc_tiling_on_sc=False),
    )
    def gather_kernel(idx_hbm, table_hbm, out_hbm, idx_v, *bufs_and_sems):
        rows = bufs_and_sems[:NBUF]
        gsem = bufs_and_sems[NBUF:]
        wid = lax.axis_index("s") * NC + lax.axis_index("c")
        base = wid * b_per_w

        pltpu.sync_copy(idx_hbm.at[pl.ds(wid * n_steps, n_steps)], idx_v)

        for b in range(NBUF):
            pltpu.async_copy(table_hbm.at[idx_v.at[b]], rows[b], gsem[b])

        def outer(i, carry):
            for b in range(NBUF):
                j = i * NBUF + b
                jn = j + NBUF

                @pl.when(j < n_steps)
                def _():
                    pltpu.make_async_copy(
                        table_hbm.at[idx_v.at[0]], rows[b], gsem[b]
                    ).wait()
                    pltpu.sync_copy(
                        rows[b], out_hbm.at[pl.ds(base + j * CHUNK, CHUNK)]
                    )

                    @pl.when(jn < n_steps)
                    def _():
                        pltpu.async_copy(
                            table_hbm.at[idx_v.at[jn]], rows[b], gsem[b]
                        )

            return carry

        lax.fori_loop(0, n_outer, outer, 0)

    return gather_kernel


def kernel(inp, table):
    b, h = inp.shape
    _, width = table.shape
    idx = inp.reshape(b * h // CHUNK, CHUNK).astype(jnp.int32)
    out = _make_gather(b * h, width)(idx, table)
    return out.reshape(b, h, width)
